# Initial kernel scaffold; baseline (speedup 1.0000x reference)
import functools

import jax
import jax.numpy as jnp
from jax import lax
from jax.experimental import pallas as pl
from jax.experimental.pallas import tpu as pltpu
from jax.experimental.pallas import tpu_sc as plsc

_N = 10000
_E = 320000
_D = 128
_DE = 16
_BN_EPS = 1e-5

_NC = 2
_NS = 16
_L = 16

_EPW = _E // (_NC * _NS)
_K = 80
_NCHUNK = _EPW // _K
_RPS = _N // _NS
_ZB = 125

_PACK = 8
_EP = _E // _PACK
_BK = 2000


def _enc_body(a_ref, k_ref, b_ref, o_ref):
    a = a_ref[...].astype(jnp.bfloat16)
    o_ref[...] = (
        jnp.dot(a, k_ref[...], preferred_element_type=jnp.float32) + b_ref[...]
    )


def _edge_encode(attr_packed, k_big, bias_tiled):
    return pl.pallas_call(
        _enc_body,
        grid=(_EP // _BK,),
        in_specs=[
            pl.BlockSpec((_BK, _D), lambda i: (i, 0)),
            pl.BlockSpec((_D, _PACK * _D), lambda i: (0, 0)),
            pl.BlockSpec((1, _PACK * _D), lambda i: (0, 0)),
        ],
        out_specs=pl.BlockSpec((_BK, _PACK * _D), lambda i: (i, 0)),
        out_shape=jax.ShapeDtypeStruct((_EP, _PACK * _D), jnp.float32),
    )(attr_packed, k_big, bias_tiled)


def _sc_body(x_hbm, ea_hbm, src_hbm, dst_hbm, out0_hbm, out1_hbm,
             srcv, dstv, xg, eav, zb, agg, sem):
    cid = lax.axis_index("c")
    sid = lax.axis_index("s")
    wid = sid * _NC + cid

    @pl.loop(0, _ZB)
    def _(r):
        for cc in range(0, _D, _L):
            zb[r, pl.ds(cc, _L)] = jnp.zeros((_L,), jnp.float32)

    @pl.loop(0, _RPS // _ZB)
    def _(j):
        pltpu.sync_copy(zb, agg.at[pl.ds(sid * _RPS + j * _ZB, _ZB)])

    plsc.subcore_barrier()

    @pl.loop(0, _NCHUNK)
    def _(c):
        base = wid * _EPW + c * _K
        pltpu.sync_copy(src_hbm.at[pl.ds(base, _K)], srcv)
        pltpu.sync_copy(dst_hbm.at[pl.ds(base, _K)], dstv)
        pltpu.async_copy(x_hbm.at[srcv], xg, sem).wait()
        pltpu.sync_copy(ea_hbm.at[pl.ds(base, _K)], eav)

        @pl.loop(0, _K)
        def _(r):
            for cc in range(0, _D, _L):
                v = xg[r, pl.ds(cc, _L)] + eav[r, pl.ds(cc, _L)]
                xg[r, pl.ds(cc, _L)] = jnp.maximum(v, 0.0)

        pltpu.sync_copy(xg, agg.at[dstv], add=True)

    plsc.subcore_barrier()

    @pl.loop(0, _RPS // _ZB)
    def _(j):
        row = sid * _RPS + j * _ZB

        @pl.when(cid == 0)
        def _():
            pltpu.sync_copy(agg.at[pl.ds(row, _ZB)], out0_hbm.at[pl.ds(row, _ZB)])

        @pl.when(cid == 1)
        def _():
            pltpu.sync_copy(agg.at[pl.ds(row, _ZB)], out1_hbm.at[pl.ds(row, _ZB)])


def _sc_aggregate(x, ea, src, dst):
    mesh = plsc.VectorSubcoreMesh(
        core_axis_name="c", subcore_axis_name="s",
        num_cores=_NC, num_subcores=_NS,
    )
    f = pl.kernel(
        _sc_body,
        out_type=[
            jax.ShapeDtypeStruct((_N, _D), jnp.float32),
            jax.ShapeDtypeStruct((_N, _D), jnp.float32),
        ],
        mesh=mesh,
        scratch_types=[
            pltpu.VMEM((_K,), jnp.int32),
            pltpu.VMEM((_K,), jnp.int32),
            pltpu.VMEM((_K, _D), jnp.float32),
            pltpu.VMEM((_K, _D), jnp.float32),
            pltpu.VMEM((_ZB, _D), jnp.float32),
            pltpu.VMEM_SHARED((_N, _D), jnp.float32),
            pltpu.SemaphoreType.DMA,
        ],
    )
    return f(x, ea, src, dst)


def _mlp_body(eps_ref, x_ref, p0_ref, p1_ref, w1_ref, c1_ref, w2_ref, c2_ref,
              s3_ref, c3_ref, o_ref):
    a = (1.0 + eps_ref[0]) * x_ref[...] + p0_ref[...] + p1_ref[...]
    h = jnp.dot(a, w1_ref[...], preferred_element_type=jnp.float32) + c1_ref[...]
    h = jnp.maximum(h, 0.0)
    h = jnp.dot(h, w2_ref[...], preferred_element_type=jnp.float32) + c2_ref[...]
    h = jnp.maximum(h, 0.0)
    o_ref[...] = jnp.maximum(h * s3_ref[...] + c3_ref[...], 0.0)


def _mlp(x, p0, p1, eps_p, w1f, c1, w2f, c2, s3, c3):
    bn = 2000
    return pl.pallas_call(
        _mlp_body,
        grid=(_N // bn,),
        in_specs=[
            pl.BlockSpec(memory_space=pltpu.SMEM),
            pl.BlockSpec((bn, _D), lambda i: (i, 0)),
            pl.BlockSpec((bn, _D), lambda i: (i, 0)),
            pl.BlockSpec((bn, _D), lambda i: (i, 0)),
            pl.BlockSpec((_D, _D), lambda i: (0, 0)),
            pl.BlockSpec((1, _D), lambda i: (0, 0)),
            pl.BlockSpec((_D, _D), lambda i: (0, 0)),
            pl.BlockSpec((1, _D), lambda i: (0, 0)),
            pl.BlockSpec((1, _D), lambda i: (0, 0)),
            pl.BlockSpec((1, _D), lambda i: (0, 0)),
        ],
        out_specs=pl.BlockSpec((bn, _D), lambda i: (i, 0)),
        out_shape=jax.ShapeDtypeStruct((_N, _D), jnp.float32),
    )(eps_p.reshape(1), x, p0, p1, w1f, c1, w2f, c2, s3, c3)


def kernel(x, edge_index, edge_attr, W_enc, b_enc, W_lin, b_lin,
           W1, b1, g1, be1, W2, b2, g2, be2, g_bn, be_bn, eps_p):
    Wc = W_lin @ W_enc
    bc = W_lin @ b_enc + b_lin
    eye = jnp.eye(_PACK, dtype=jnp.float32)
    k_big = jnp.einsum("ab,jo->ajbo", eye, Wc.T)
    k_big = k_big.reshape(_PACK * _DE, _PACK * _D).astype(jnp.bfloat16)
    bias_tiled = jnp.tile(bc, _PACK).reshape(1, _PACK * _D)

    inv = 1.0 / jnp.sqrt(1.0 + _BN_EPS)
    w1f = W1.T * (inv * g1)[None, :]
    c1 = ((b1 * inv) * g1 + be1).reshape(1, _D)
    w2f = W2.T * (inv * g2)[None, :]
    c2 = ((b2 * inv) * g2 + be2).reshape(1, _D)
    s3 = (inv * g_bn).reshape(1, _D)
    c3 = be_bn.reshape(1, _D)

    attr_packed = edge_attr.reshape(_EP, _PACK * _DE)
    ea = _edge_encode(attr_packed, k_big, bias_tiled).reshape(_E, _D)

    src = edge_index[0]
    dst = edge_index[1]
    p0, p1 = _sc_aggregate(x, ea, src, dst)

    return _mlp(x, p0, p1, eps_p, w1f, c1, w2f, c2, s3, c3)


# speedup vs baseline: 2.1054x; 2.1054x over previous
import functools

import jax
import jax.numpy as jnp
from jax import lax
from jax.experimental import pallas as pl
from jax.experimental.pallas import tpu as pltpu
from jax.experimental.pallas import tpu_sc as plsc

_N = 10000
_E = 320000
_D = 128
_DE = 16
_BN_EPS = 1e-5

_NC = 2
_NS = 16
_L = 16

_EPW = _E // (_NC * _NS)
_K = 80
_NCHUNK = _EPW // _K
_NP = 10240
_RPS = _NP // _NS
_ZB = 128

_PACK = 8
_EP = _E // _PACK
_BK = 2000


def _enc_body(a_ref, k_ref, b_ref, o_ref):
    a = a_ref[...].astype(jnp.bfloat16)
    o_ref[...] = (
        jnp.dot(a, k_ref[...], preferred_element_type=jnp.float32) + b_ref[...]
    )


def _edge_encode(attr_packed, k_big, bias_tiled):
    return pl.pallas_call(
        _enc_body,
        grid=(_EP // _BK,),
        in_specs=[
            pl.BlockSpec((_BK, _D), lambda i: (i, 0)),
            pl.BlockSpec((_D, _PACK * _D), lambda i: (0, 0)),
            pl.BlockSpec((1, _PACK * _D), lambda i: (0, 0)),
        ],
        out_specs=pl.BlockSpec((_BK, _PACK * _D), lambda i: (i, 0)),
        out_shape=jax.ShapeDtypeStruct((_EP, _PACK * _D), jnp.float32),
    )(attr_packed, k_big, bias_tiled)


def _sc_body(x_hbm, ea_hbm, src_hbm, dst_hbm, out0_hbm, out1_hbm,
             srcv, dstv, xg, eav, zb, agg, sem):
    cid = lax.axis_index("c")
    sid = lax.axis_index("s")
    wid = sid * _NC + cid

    @pl.loop(0, _ZB)
    def _(r):
        for cc in range(0, _D, _L):
            zb[r, pl.ds(cc, _L)] = jnp.zeros((_L,), jnp.float32)

    @pl.loop(0, _RPS // _ZB)
    def _(j):
        pltpu.sync_copy(zb, agg.at[pl.ds(sid * _RPS + j * _ZB, _ZB)])

    plsc.subcore_barrier()

    @pl.loop(0, _NCHUNK)
    def _(c):
        base = wid * _EPW + c * _K
        pltpu.sync_copy(src_hbm.at[pl.ds(base, _K)], srcv)
        pltpu.sync_copy(dst_hbm.at[pl.ds(base, _K)], dstv)
        pltpu.async_copy(x_hbm.at[srcv], xg, sem).wait()
        pltpu.sync_copy(ea_hbm.at[pl.ds(base, _K)], eav)

        @pl.loop(0, _K)
        def _(r):
            for cc in range(0, _D, _L):
                v = xg[r, pl.ds(cc, _L)] + eav[r, pl.ds(cc, _L)]
                xg[r, pl.ds(cc, _L)] = jnp.maximum(v, 0.0)

        pltpu.sync_copy(xg, agg.at[dstv], add=True)

    plsc.subcore_barrier()

    @pl.loop(0, _RPS // _ZB)
    def _(j):
        row = sid * _RPS + j * _ZB

        @pl.when(cid == 0)
        def _():
            pltpu.sync_copy(agg.at[pl.ds(row, _ZB)], out0_hbm.at[pl.ds(row, _ZB)])

        @pl.when(cid == 1)
        def _():
            pltpu.sync_copy(agg.at[pl.ds(row, _ZB)], out1_hbm.at[pl.ds(row, _ZB)])


def _sc_aggregate(x, ea, src, dst):
    mesh = plsc.VectorSubcoreMesh(
        core_axis_name="c", subcore_axis_name="s",
        num_cores=_NC, num_subcores=_NS,
    )
    f = pl.kernel(
        _sc_body,
        out_type=[
            jax.ShapeDtypeStruct((_NP, _D), jnp.float32),
            jax.ShapeDtypeStruct((_NP, _D), jnp.float32),
        ],
        mesh=mesh,
        scratch_types=[
            pltpu.VMEM((_K,), jnp.int32),
            pltpu.VMEM((_K,), jnp.int32),
            pltpu.VMEM((_K, _D), jnp.float32),
            pltpu.VMEM((_K, _D), jnp.float32),
            pltpu.VMEM((_ZB, _D), jnp.float32),
            pltpu.VMEM_SHARED((_NP, _D), jnp.float32),
            pltpu.SemaphoreType.DMA,
        ],
    )
    return f(x, ea, src, dst)


def _mlp_body(eps_ref, x_ref, p0_ref, p1_ref, w1_ref, c1_ref, w2_ref, c2_ref,
              s3_ref, c3_ref, o_ref):
    a = (1.0 + eps_ref[0]) * x_ref[...] + p0_ref[...] + p1_ref[...]
    h = jnp.dot(a, w1_ref[...], preferred_element_type=jnp.float32) + c1_ref[...]
    h = jnp.maximum(h, 0.0)
    h = jnp.dot(h, w2_ref[...], preferred_element_type=jnp.float32) + c2_ref[...]
    h = jnp.maximum(h, 0.0)
    o_ref[...] = jnp.maximum(h * s3_ref[...] + c3_ref[...], 0.0)


def _mlp(x, p0, p1, eps_p, w1f, c1, w2f, c2, s3, c3):
    bn = 2000
    return pl.pallas_call(
        _mlp_body,
        grid=(_N // bn,),
        in_specs=[
            pl.BlockSpec(memory_space=pltpu.SMEM),
            pl.BlockSpec((bn, _D), lambda i: (i, 0)),
            pl.BlockSpec((bn, _D), lambda i: (i, 0)),
            pl.BlockSpec((bn, _D), lambda i: (i, 0)),
            pl.BlockSpec((_D, _D), lambda i: (0, 0)),
            pl.BlockSpec((1, _D), lambda i: (0, 0)),
            pl.BlockSpec((_D, _D), lambda i: (0, 0)),
            pl.BlockSpec((1, _D), lambda i: (0, 0)),
            pl.BlockSpec((1, _D), lambda i: (0, 0)),
            pl.BlockSpec((1, _D), lambda i: (0, 0)),
        ],
        out_specs=pl.BlockSpec((bn, _D), lambda i: (i, 0)),
        out_shape=jax.ShapeDtypeStruct((_N, _D), jnp.float32),
    )(eps_p.reshape(1), x, p0, p1, w1f, c1, w2f, c2, s3, c3)


def kernel(x, edge_index, edge_attr, W_enc, b_enc, W_lin, b_lin,
           W1, b1, g1, be1, W2, b2, g2, be2, g_bn, be_bn, eps_p):
    Wc = W_lin @ W_enc
    bc = W_lin @ b_enc + b_lin
    eye = jnp.eye(_PACK, dtype=jnp.float32)
    k_big = jnp.einsum("ab,jo->ajbo", eye, Wc.T)
    k_big = k_big.reshape(_PACK * _DE, _PACK * _D).astype(jnp.bfloat16)
    bias_tiled = jnp.tile(bc, _PACK).reshape(1, _PACK * _D)

    inv = 1.0 / jnp.sqrt(1.0 + _BN_EPS)
    w1f = W1.T * (inv * g1)[None, :]
    c1 = ((b1 * inv) * g1 + be1).reshape(1, _D)
    w2f = W2.T * (inv * g2)[None, :]
    c2 = ((b2 * inv) * g2 + be2).reshape(1, _D)
    s3 = (inv * g_bn).reshape(1, _D)
    c3 = be_bn.reshape(1, _D)

    attr_packed = edge_attr.reshape(_EP, _PACK * _DE)
    ea = _edge_encode(attr_packed, k_big, bias_tiled).reshape(_E, _D)

    src = edge_index[0]
    dst = edge_index[1]
    p0, p1 = _sc_aggregate(x, ea, src, dst)

    return _mlp(x, p0, p1, eps_p, w1f, c1, w2f, c2, s3, c3)
